# baseline (speedup 1.0000x reference)
import functools

import numpy as np
import jax
import jax.numpy as jnp
from jax.experimental import pallas as pl
from jax.experimental.pallas import tpu as pltpu

_SCALES = (1, 2, 4, 7, 14, 28)
_NC = 8192
_H = 28
_HW = _H * _H
_HIGH = jax.lax.Precision.HIGHEST


def _pool_A(s):
    k = _H // s
    P = np.zeros((s, _H), np.float32)
    for i in range(s):
        P[i, i * k:(i + 1) * k] = np.float32(1.0 / k)
    return np.einsum('iy,jx->ijyx', P, P).reshape(s * s, _HW).copy()


@functools.lru_cache(maxsize=None)
def _pool_mats():
    return {s: _pool_A(s) for s in _SCALES}


def _vq_chunk(tok, C, cn):
    zn = jnp.sum(tok * tok, axis=1, keepdims=True)
    mm = jax.lax.dot_general(tok, C, (((1,), (1,)), ((), ())),
                             preferred_element_type=jnp.float32)
    d = (zn + cn) - 2.0 * mm
    m = jnp.min(d, axis=1, keepdims=True)
    iota = jax.lax.broadcasted_iota(jnp.int32, d.shape, 1)
    idx = jnp.min(jnp.where(d == m, iota, jnp.int32(_NC)), axis=1, keepdims=True)
    oh = (iota == idx).astype(jnp.float32)
    zq = jax.lax.dot_general(oh, C, (((1,), (0,)), ((), ())),
                             precision=_HIGH, preferred_element_type=jnp.float32)
    cnts = jax.lax.dot_general(jnp.ones((1, tok.shape[0]), jnp.float32), oh,
                               (((1,), (0,)), ((), ())),
                               precision=_HIGH, preferred_element_type=jnp.float32)
    e = (zq - tok) ** 2
    sse = jnp.sum(jnp.sum(e, axis=1, keepdims=True), axis=0, keepdims=True)
    return zq, cnts, sse


def _vq_scale_body(s, f_ref, cb_ref, a_ref, zq_ref, vl_ref, perp_ref, tok_ref):
    T = s * s
    n = 2 * T
    C = cb_ref[...]
    cn = jax.lax.dot_general(jnp.ones((1, 128), jnp.float32), C * C,
                             (((1,), (1,)), ((), ())),
                             precision=_HIGH, preferred_element_type=jnp.float32)
    A = a_ref[...]
    t0 = jnp.dot(A, f_ref[0], precision=_HIGH, preferred_element_type=jnp.float32)
    t1 = jnp.dot(A, f_ref[1], precision=_HIGH, preferred_element_type=jnp.float32)
    toks = jnp.concatenate([t0, t1], axis=0)

    if s == 28:
        tok_ref[...] = toks
        chunk = 224

        def body(j, carry):
            cnts_c, sse_c = carry
            tk = tok_ref[pl.ds(j * chunk, chunk), :]
            zq_c, c_c, s_c = _vq_chunk(tk, C, cn)
            zq_ref[pl.ds(j * chunk, chunk), :] = zq_c
            return (cnts_c + c_c, sse_c + s_c)

        cnts, sse = jax.lax.fori_loop(
            0, n // chunk, body,
            (jnp.zeros((1, _NC), jnp.float32), jnp.zeros((1, 1), jnp.float32)))
    elif s == 14:
        h = n // 2
        zq_a, cnts_a, sse_a = _vq_chunk(toks[:h], C, cn)
        zq_b, cnts_b, sse_b = _vq_chunk(toks[h:], C, cn)
        zq_ref[...] = jnp.concatenate([zq_a, zq_b], axis=0)
        cnts = cnts_a + cnts_b
        sse = sse_a + sse_b
    else:
        zq_all, cnts, sse = _vq_chunk(toks, C, cn)
        zq_ref[...] = zq_all

    avg = cnts * jnp.float32(1.0 / n)
    ent = jnp.sum(avg * jnp.log(avg + 1e-10), axis=1, keepdims=True)
    perp_ref[...] = jnp.exp(-ent)
    mse = sse * jnp.float32(1.0 / (n * 128))
    vl_ref[...] = mse + 0.25 * mse


@functools.lru_cache(maxsize=None)
def _vq_scale_call(s):
    T = s * s
    body = functools.partial(_vq_scale_body, s)
    return pl.pallas_call(
        body,
        out_shape=[
            jax.ShapeDtypeStruct((2 * T, 128), jnp.float32),
            jax.ShapeDtypeStruct((1, 1), jnp.float32),
            jax.ShapeDtypeStruct((1, 1), jnp.float32),
        ],
        scratch_shapes=[pltpu.VMEM((2 * T, 128), jnp.float32)],
    )


def _conv2d(x, w, b=None, stride=1, pad=0):
    out = jax.lax.conv_general_dilated(x, jnp.transpose(w, (2, 3, 1, 0)),
                                       (stride, stride), [(pad, pad), (pad, pad)],
                                       dimension_numbers=('NHWC', 'HWIO', 'NHWC'))
    if b is not None:
        out = out + b[None, None, None, :]
    return out


def _conv_transpose2d(x, w, b, stride=2, pad=1):
    k = w.shape[2]
    wt = jnp.transpose(jnp.flip(w, axis=(2, 3)), (2, 3, 0, 1))
    e = k - 1 - pad
    out = jax.lax.conv_general_dilated(x, wt, (1, 1), [(e, e), (e, e)],
                                       lhs_dilation=(stride, stride),
                                       dimension_numbers=('NHWC', 'HWIO', 'NHWC'))
    return out + b[None, None, None, :]


def _bn_eval(x, g, b, eps=1e-5):
    return x / np.float32(np.sqrt(1.0 + eps)) * g[None, None, None, :] + b[None, None, None, :]


def _encoder(xf, p):
    h = _conv2d(xf, p['enc_comp_w'], None, 1, 1)
    h = jax.nn.relu(_bn_eval(h, p['enc_bn_g'], p['enc_bn_b']))
    h = jax.nn.relu(_conv2d(h, p['enc_w1'], p['enc_b1'], 2, 1))
    h = jax.nn.relu(_conv2d(h, p['enc_w2'], p['enc_b2'], 2, 1))
    h = jax.nn.relu(_conv2d(h, p['enc_w3'], p['enc_b3'], 2, 1))
    h = _conv2d(h, p['enc_w4'], p['enc_b4'], 1, 1)
    return h


def _decoder(f, p):
    h = jax.nn.relu(_conv2d(f, p['dec_w0'], p['dec_b0'], 1, 1))
    h = jax.nn.relu(_conv_transpose2d(h, p['dct_w1'], p['dct_b1']))
    h = jax.nn.relu(_conv_transpose2d(h, p['dct_w2'], p['dct_b2']))
    h = _conv_transpose2d(h, p['dct_w3'], p['dct_b3'])
    h = _conv2d(h, p['decomp_w'], p['decomp_b'], 1, 0)
    return h


def kernel(x, params):
    p = params
    B = x.shape[0]
    xT = jnp.transpose(x, (0, 2, 3, 1))
    xf = jax.nn.one_hot(xT, 18, dtype=jnp.float32).reshape(B, 224, 224, 72)
    f = _encoder(xf, p)
    cb = p['codebook']

    with jax.ensure_compile_time_eval():
        A_mats = _pool_mats()

    vq_loss = jnp.asarray(0.0, jnp.float32)
    perps = []
    fd = jnp.zeros_like(f)
    for i, s in enumerate(_SCALES):
        f_tok = f.reshape(B, _HW, 128)
        zq, vl, perp = _vq_scale_call(s)(f_tok, cb, jnp.asarray(A_mats[s]))
        vq_loss = vq_loss + vl[0, 0]
        perps.append(perp[0, 0])
        z = zq.reshape(B, s, s, 128)
        z = jax.image.resize(z, (B, _H, _H, 128), method='cubic')
        f = f - jax.nn.relu(_conv2d(z, p['phi_enc_w'][i], p['phi_enc_b'][i], 1, 1))
        fd = fd + jax.nn.relu(_conv2d(z, p['phi_dec_w'][i], p['phi_dec_b'][i], 1, 1))

    xh = _decoder(fd, p)
    recon = jnp.mean(jnp.maximum(xh, 0.0) - xh * xf + jnp.log1p(jnp.exp(-jnp.abs(xh))))
    x_hat = jnp.transpose(xh, (0, 3, 1, 2))
    return x_hat, recon, vq_loss, jnp.stack(perps)


# speedup vs baseline: 1.4760x; 1.4760x over previous
import functools

import numpy as np
import jax
import jax.numpy as jnp
from jax.experimental import pallas as pl
from jax.experimental.pallas import tpu as pltpu

_SCALES = (1, 2, 4, 7, 14, 28)
_NC = 8192
_H = 28
_HW = _H * _H
_HIGH = jax.lax.Precision.HIGHEST


def _pool_A(s):
    k = _H // s
    P = np.zeros((s, _H), np.float32)
    for i in range(s):
        P[i, i * k:(i + 1) * k] = np.float32(1.0 / k)
    return np.einsum('iy,jx->ijyx', P, P).reshape(s * s, _HW).copy()


@functools.lru_cache(maxsize=None)
def _pool_mats():
    return {s: _pool_A(s) for s in _SCALES}


def _vq_chunk(tok, C, cn):
    zn = jnp.sum(tok * tok, axis=1, keepdims=True)
    mm = jax.lax.dot_general(tok, C, (((1,), (1,)), ((), ())),
                             preferred_element_type=jnp.float32)
    d = (zn + cn) - 2.0 * mm
    m = jnp.min(d, axis=1, keepdims=True)
    iota = jax.lax.broadcasted_iota(jnp.int32, d.shape, 1)
    idx = jnp.min(jnp.where(d == m, iota, jnp.int32(_NC)), axis=1, keepdims=True)
    oh = (iota == idx).astype(jnp.float32)
    zq = jax.lax.dot_general(oh, C, (((1,), (0,)), ((), ())),
                             precision=_HIGH, preferred_element_type=jnp.float32)
    cnts = jax.lax.dot_general(jnp.ones((1, tok.shape[0]), jnp.float32), oh,
                               (((1,), (0,)), ((), ())),
                               precision=_HIGH, preferred_element_type=jnp.float32)
    e = (zq - tok) ** 2
    sse = jnp.sum(jnp.sum(e, axis=1, keepdims=True), axis=0, keepdims=True)
    return zq, cnts, sse


def _vq_scale_body(s, f_ref, cb_ref, a_ref, zq_ref, vl_ref, perp_ref, tok_ref):
    T = s * s
    n = 2 * T
    C = cb_ref[...]
    cn = jax.lax.dot_general(jnp.ones((1, 128), jnp.float32), C * C,
                             (((1,), (1,)), ((), ())),
                             precision=_HIGH, preferred_element_type=jnp.float32)
    A = a_ref[...]
    t0 = jnp.dot(A, f_ref[0], precision=_HIGH, preferred_element_type=jnp.float32)
    t1 = jnp.dot(A, f_ref[1], precision=_HIGH, preferred_element_type=jnp.float32)
    toks = jnp.concatenate([t0, t1], axis=0)

    if s == 28:
        tok_ref[...] = toks
        chunk = 224

        def body(j, carry):
            cnts_c, sse_c = carry
            tk = tok_ref[pl.ds(j * chunk, chunk), :]
            zq_c, c_c, s_c = _vq_chunk(tk, C, cn)
            zq_ref[pl.ds(j * chunk, chunk), :] = zq_c
            return (cnts_c + c_c, sse_c + s_c)

        cnts, sse = jax.lax.fori_loop(
            0, n // chunk, body,
            (jnp.zeros((1, _NC), jnp.float32), jnp.zeros((1, 1), jnp.float32)))
    elif s == 14:
        h = n // 2
        zq_a, cnts_a, sse_a = _vq_chunk(toks[:h], C, cn)
        zq_b, cnts_b, sse_b = _vq_chunk(toks[h:], C, cn)
        zq_ref[...] = jnp.concatenate([zq_a, zq_b], axis=0)
        cnts = cnts_a + cnts_b
        sse = sse_a + sse_b
    else:
        zq_all, cnts, sse = _vq_chunk(toks, C, cn)
        zq_ref[...] = zq_all

    avg = cnts * jnp.float32(1.0 / n)
    ent = jnp.sum(avg * jnp.log(avg + 1e-10), axis=1, keepdims=True)
    perp_ref[...] = jnp.exp(-ent)
    mse = sse * jnp.float32(1.0 / (n * 128))
    vl_ref[...] = mse + 0.25 * mse


@functools.lru_cache(maxsize=None)
def _vq_scale_call(s):
    T = s * s
    body = functools.partial(_vq_scale_body, s)
    return pl.pallas_call(
        body,
        out_shape=[
            jax.ShapeDtypeStruct((2 * T, 128), jnp.float32),
            jax.ShapeDtypeStruct((1, 1), jnp.float32),
            jax.ShapeDtypeStruct((1, 1), jnp.float32),
        ],
        scratch_shapes=[pltpu.VMEM((2 * T, 128), jnp.float32)],
    )


_PIX = 224 * 224
_TBLK = 3584


def _tail_body(h_ref, x_ref, w_ref, b_ref, xhat_ref, acc_ref):
    v = jax.lax.dot_general(w_ref[...], h_ref[0], (((1,), (0,)), ((), ())),
                            preferred_element_type=jnp.float32) + b_ref[...]
    xhat_ref[0] = v
    pw = jnp.maximum(v, 0.0) + jnp.log1p(jnp.exp(-jnp.abs(v)))
    psum = jnp.sum(jnp.sum(pw, axis=1, keepdims=True), axis=0, keepdims=True)
    chan = jax.lax.broadcasted_iota(jnp.int32, v.shape, 0)
    gsum = jnp.zeros((1, 1), jnp.float32)
    for z in range(4):
        lane = x_ref[0, z:z + 1, :] + jnp.int32(18 * z)
        sel = jnp.where(chan == lane, v, 0.0)
        gsum = gsum + jnp.sum(jnp.sum(sel, axis=1, keepdims=True), axis=0, keepdims=True)

    i = pl.program_id(0)
    j = pl.program_id(1)

    @pl.when(jnp.logical_and(i == 0, j == 0))
    def _():
        acc_ref[...] = jnp.zeros_like(acc_ref)

    acc_ref[...] = acc_ref[...] + jnp.concatenate([psum, gsum], axis=1)


@functools.lru_cache(maxsize=None)
def _tail_call():
    nblk = _PIX // _TBLK
    return pl.pallas_call(
        _tail_body,
        grid=(2, nblk),
        in_specs=[
            pl.BlockSpec((1, 64, _TBLK), lambda i, j: (i, 0, j)),
            pl.BlockSpec((1, 4, _TBLK), lambda i, j: (i, 0, j)),
            pl.BlockSpec((72, 64), lambda i, j: (0, 0)),
            pl.BlockSpec((72, 1), lambda i, j: (0, 0)),
        ],
        out_specs=[
            pl.BlockSpec((1, 72, _TBLK), lambda i, j: (i, 0, j)),
            pl.BlockSpec((1, 2), lambda i, j: (0, 0)),
        ],
        out_shape=[
            jax.ShapeDtypeStruct((2, 72, _PIX), jnp.float32),
            jax.ShapeDtypeStruct((1, 2), jnp.float32),
        ],
    )


def _conv2d(x, w, b=None, stride=1, pad=0):
    out = jax.lax.conv_general_dilated(x, w, (stride, stride), [(pad, pad), (pad, pad)],
                                       dimension_numbers=('NCHW', 'OIHW', 'NCHW'))
    if b is not None:
        out = out + b[None, :, None, None]
    return out


def _conv_transpose2d(x, w, b, stride=2, pad=1):
    k = w.shape[2]
    wt = jnp.flip(w, axis=(2, 3)).transpose(1, 0, 2, 3)
    e = k - 1 - pad
    out = jax.lax.conv_general_dilated(x, wt, (1, 1), [(e, e), (e, e)],
                                       lhs_dilation=(stride, stride),
                                       dimension_numbers=('NCHW', 'OIHW', 'NCHW'))
    return out + b[None, :, None, None]


def _bn_eval(x, g, b, eps=1e-5):
    return x / np.float32(np.sqrt(1.0 + eps)) * g[None, :, None, None] + b[None, :, None, None]


def _encoder(xf, p):
    h = _conv2d(xf, p['enc_comp_w'], None, 1, 1)
    h = jax.nn.relu(_bn_eval(h, p['enc_bn_g'], p['enc_bn_b']))
    h = jax.nn.relu(_conv2d(h, p['enc_w1'], p['enc_b1'], 2, 1))
    h = jax.nn.relu(_conv2d(h, p['enc_w2'], p['enc_b2'], 2, 1))
    h = jax.nn.relu(_conv2d(h, p['enc_w3'], p['enc_b3'], 2, 1))
    h = _conv2d(h, p['enc_w4'], p['enc_b4'], 1, 1)
    return h


def kernel(x, params):
    p = params
    B = x.shape[0]
    x_oh = jax.nn.one_hot(x, 18, dtype=jnp.float32)
    xf = x_oh.transpose(0, 1, 4, 2, 3).reshape(B, x.shape[1] * 18, x.shape[2], x.shape[3])
    f = _encoder(xf, p)
    cb = p['codebook']

    with jax.ensure_compile_time_eval():
        A_mats = _pool_mats()

    vq_loss = jnp.asarray(0.0, jnp.float32)
    perps = []
    fd = jnp.zeros_like(f)
    for i, s in enumerate(_SCALES):
        f_tok = jnp.transpose(f, (0, 2, 3, 1)).reshape(B, _HW, 128)
        zq, vl, perp = _vq_scale_call(s)(f_tok, cb, jnp.asarray(A_mats[s]))
        vq_loss = vq_loss + vl[0, 0]
        perps.append(perp[0, 0])
        z = zq.reshape(B, s, s, 128).transpose(0, 3, 1, 2)
        z = jax.image.resize(z, (B, 128, _H, _H), method='cubic')
        f = f - jax.nn.relu(_conv2d(z, p['phi_enc_w'][i], p['phi_enc_b'][i], 1, 1))
        fd = fd + jax.nn.relu(_conv2d(z, p['phi_dec_w'][i], p['phi_dec_b'][i], 1, 1))

    h = jax.nn.relu(_conv2d(fd, p['dec_w0'], p['dec_b0'], 1, 1))
    h = jax.nn.relu(_conv_transpose2d(h, p['dct_w1'], p['dct_b1']))
    h = jax.nn.relu(_conv_transpose2d(h, p['dct_w2'], p['dct_b2']))
    h = _conv_transpose2d(h, p['dct_w3'], p['dct_b3'])

    w_oc = p['decomp_w'][:, :, 0, 0]
    xh, acc = _tail_call()(h.reshape(B, 64, _PIX), x.reshape(B, 4, _PIX),
                           w_oc, p['decomp_b'][:, None])
    x_hat = xh.reshape(B, 72, 224, 224)
    recon = (acc[0, 0] - acc[0, 1]) * jnp.float32(1.0 / (B * 72 * _PIX))
    return x_hat, recon, vq_loss, jnp.stack(perps)


# speedup vs baseline: 1.7581x; 1.1911x over previous
import functools

import numpy as np
import jax
import jax.numpy as jnp
from jax.experimental import pallas as pl
from jax.experimental.pallas import tpu as pltpu

_SCALES = (1, 2, 4, 7, 14, 28)
_NC = 8192
_H = 28
_HW = _H * _H
_HIGH = jax.lax.Precision.HIGHEST


def _pool_A(s):
    k = _H // s
    P = np.zeros((s, _H), np.float32)
    for i in range(s):
        P[i, i * k:(i + 1) * k] = np.float32(1.0 / k)
    return np.einsum('iy,jx->ijyx', P, P).reshape(s * s, _HW).copy()


@functools.lru_cache(maxsize=None)
def _pool_mats():
    return {s: _pool_A(s) for s in _SCALES}


def _vq_chunk(tok, C, cn):
    zn = jnp.sum(tok * tok, axis=1, keepdims=True)
    mm = jax.lax.dot_general(tok, C, (((1,), (1,)), ((), ())),
                             preferred_element_type=jnp.float32)
    d = (zn + cn) - 2.0 * mm
    m = jnp.min(d, axis=1, keepdims=True)
    iota = jax.lax.broadcasted_iota(jnp.int32, d.shape, 1)
    idx = jnp.min(jnp.where(d == m, iota, jnp.int32(_NC)), axis=1, keepdims=True)
    oh = (iota == idx).astype(jnp.float32)
    zq = jax.lax.dot_general(oh, C, (((1,), (0,)), ((), ())),
                             precision=_HIGH, preferred_element_type=jnp.float32)
    cnts = jax.lax.dot_general(jnp.ones((1, tok.shape[0]), jnp.float32), oh,
                               (((1,), (0,)), ((), ())),
                               precision=_HIGH, preferred_element_type=jnp.float32)
    e = (zq - tok) ** 2
    sse = jnp.sum(jnp.sum(e, axis=1, keepdims=True), axis=0, keepdims=True)
    return zq, cnts, sse


def _vq_scale_body(s, f_ref, cb_ref, a_ref, zq_ref, vl_ref, perp_ref, tok_ref):
    T = s * s
    n = 2 * T
    C = cb_ref[...]
    cn = jax.lax.dot_general(jnp.ones((1, 128), jnp.float32), C * C,
                             (((1,), (1,)), ((), ())),
                             precision=_HIGH, preferred_element_type=jnp.float32)
    A = a_ref[...]
    t0 = jnp.dot(A, f_ref[0], precision=_HIGH, preferred_element_type=jnp.float32)
    t1 = jnp.dot(A, f_ref[1], precision=_HIGH, preferred_element_type=jnp.float32)
    toks = jnp.concatenate([t0, t1], axis=0)

    if s == 28:
        tok_ref[...] = toks
        chunk = 224

        def body(j, carry):
            cnts_c, sse_c = carry
            tk = tok_ref[pl.ds(j * chunk, chunk), :]
            zq_c, c_c, s_c = _vq_chunk(tk, C, cn)
            zq_ref[pl.ds(j * chunk, chunk), :] = zq_c
            return (cnts_c + c_c, sse_c + s_c)

        cnts, sse = jax.lax.fori_loop(
            0, n // chunk, body,
            (jnp.zeros((1, _NC), jnp.float32), jnp.zeros((1, 1), jnp.float32)))
    elif s == 14:
        h = n // 2
        zq_a, cnts_a, sse_a = _vq_chunk(toks[:h], C, cn)
        zq_b, cnts_b, sse_b = _vq_chunk(toks[h:], C, cn)
        zq_ref[...] = jnp.concatenate([zq_a, zq_b], axis=0)
        cnts = cnts_a + cnts_b
        sse = sse_a + sse_b
    else:
        zq_all, cnts, sse = _vq_chunk(toks, C, cn)
        zq_ref[...] = zq_all

    avg = cnts * jnp.float32(1.0 / n)
    ent = jnp.sum(avg * jnp.log(avg + 1e-10), axis=1, keepdims=True)
    perp_ref[...] = jnp.exp(-ent)
    mse = sse * jnp.float32(1.0 / (n * 128))
    vl_ref[...] = mse + 0.25 * mse


@functools.lru_cache(maxsize=None)
def _vq_scale_call(s):
    T = s * s
    body = functools.partial(_vq_scale_body, s)
    return pl.pallas_call(
        body,
        out_shape=[
            jax.ShapeDtypeStruct((2 * T, 128), jnp.float32),
            jax.ShapeDtypeStruct((1, 1), jnp.float32),
            jax.ShapeDtypeStruct((1, 1), jnp.float32),
        ],
        scratch_shapes=[pltpu.VMEM((2 * T, 128), jnp.float32)],
    )


_PIX = 224 * 224
_TBLK = 3584


def _tail_body(h_ref, x_ref, w_ref, b_ref, xhat_ref, acc_ref):
    v = jax.lax.dot_general(w_ref[...], h_ref[0], (((1,), (0,)), ((), ())),
                            preferred_element_type=jnp.float32) + b_ref[...]
    xhat_ref[0] = v
    pw = jnp.maximum(v, 0.0) + jnp.log1p(jnp.exp(-jnp.abs(v)))
    psum = jnp.sum(jnp.sum(pw, axis=1, keepdims=True), axis=0, keepdims=True)
    chan = jax.lax.broadcasted_iota(jnp.int32, v.shape, 0)
    gsum = jnp.zeros((1, 1), jnp.float32)
    for z in range(4):
        lane = x_ref[0, z:z + 1, :] + jnp.int32(18 * z)
        sel = jnp.where(chan == lane, v, 0.0)
        gsum = gsum + jnp.sum(jnp.sum(sel, axis=1, keepdims=True), axis=0, keepdims=True)

    i = pl.program_id(0)
    j = pl.program_id(1)

    @pl.when(jnp.logical_and(i == 0, j == 0))
    def _():
        acc_ref[...] = jnp.zeros_like(acc_ref)

    acc_ref[...] = acc_ref[...] + jnp.concatenate([psum, gsum], axis=1)


@functools.lru_cache(maxsize=None)
def _tail_call():
    nblk = _PIX // _TBLK
    return pl.pallas_call(
        _tail_body,
        grid=(2, nblk),
        in_specs=[
            pl.BlockSpec((1, 64, _TBLK), lambda i, j: (i, 0, j)),

            pl.BlockSpec((1, 4, _TBLK), lambda i, j: (i, 0, j)),
            pl.BlockSpec((72, 64), lambda i, j: (0, 0)),
            pl.BlockSpec((72, 1), lambda i, j: (0, 0)),
        ],
        out_specs=[
            pl.BlockSpec((1, 72, _TBLK), lambda i, j: (i, 0, j)),
            pl.BlockSpec((1, 2), lambda i, j: (0, 0)),
        ],
        out_shape=[
            jax.ShapeDtypeStruct((2, 72, _PIX), jnp.float32),
            jax.ShapeDtypeStruct((1, 2), jnp.float32),
        ],
    )


def _conv2d(x, w, b=None, stride=1, pad=0):
    out = jax.lax.conv_general_dilated(x, w, (stride, stride), [(pad, pad), (pad, pad)],
                                       dimension_numbers=('NCHW', 'OIHW', 'NCHW'))
    if b is not None:
        out = out + b[None, :, None, None]
    return out


def _conv_transpose2d(x, w, b, stride=2, pad=1):
    k = w.shape[2]
    wt = jnp.flip(w, axis=(2, 3)).transpose(1, 0, 2, 3)
    e = k - 1 - pad
    out = jax.lax.conv_general_dilated(x, wt, (1, 1), [(e, e), (e, e)],
                                       lhs_dilation=(stride, stride),
                                       dimension_numbers=('NCHW', 'OIHW', 'NCHW'))
    return out + b[None, :, None, None]


def _bn_eval(x, g, b, eps=1e-5):
    return x / np.float32(np.sqrt(1.0 + eps)) * g[None, :, None, None] + b[None, :, None, None]


def _encoder(xf, p):
    h = _conv2d(xf, p['enc_comp_w'], None, 1, 1)
    h = jax.nn.relu(_bn_eval(h, p['enc_bn_g'], p['enc_bn_b']))
    h = jax.nn.relu(_conv2d(h, p['enc_w1'], p['enc_b1'], 2, 1))
    h = jax.nn.relu(_conv2d(h, p['enc_w2'], p['enc_b2'], 2, 1))
    h = jax.nn.relu(_conv2d(h, p['enc_w3'], p['enc_b3'], 2, 1))
    h = _conv2d(h, p['enc_w4'], p['enc_b4'], 1, 1)
    return h


def kernel(x, params):
    p = params
    B = x.shape[0]
    x_oh = jax.nn.one_hot(x, 18, dtype=jnp.float32)
    xf = x_oh.transpose(0, 1, 4, 2, 3).reshape(B, x.shape[1] * 18, x.shape[2], x.shape[3])
    f = _encoder(xf, p)
    cb = p['codebook']

    with jax.ensure_compile_time_eval():
        A_mats = _pool_mats()

    vq_loss = jnp.asarray(0.0, jnp.float32)
    perps = []
    fd = jnp.zeros_like(f)
    for i, s in enumerate(_SCALES):
        f_tok = jnp.transpose(f, (0, 2, 3, 1)).reshape(B, _HW, 128)
        zq, vl, perp = _vq_scale_call(s)(f_tok, cb, jnp.asarray(A_mats[s]))
        vq_loss = vq_loss + vl[0, 0]
        perps.append(perp[0, 0])
        z = zq.reshape(B, s, s, 128).transpose(0, 3, 1, 2)
        z = jax.image.resize(z, (B, 128, _H, _H), method='cubic')
        f = f - jax.nn.relu(_conv2d(z, p['phi_enc_w'][i], p['phi_enc_b'][i], 1, 1))
        fd = fd + jax.nn.relu(_conv2d(z, p['phi_dec_w'][i], p['phi_dec_b'][i], 1, 1))

    h = jax.nn.relu(_conv2d(fd, p['dec_w0'], p['dec_b0'], 1, 1))
    h = jax.nn.relu(_conv_transpose2d(h, p['dct_w1'], p['dct_b1']))
    h = jax.nn.relu(_conv_transpose2d(h, p['dct_w2'], p['dct_b2']))
    h = _conv_transpose2d(h, p['dct_w3'], p['dct_b3'])

    w_oc = p['decomp_w'][:, :, 0, 0]
    h16 = h.astype(jnp.bfloat16)
    xh, acc = _tail_call()(h16.reshape(B, 64, _PIX), x.reshape(B, 4, _PIX),
                           w_oc, p['decomp_b'][:, None])
    x_hat = xh.reshape(B, 72, 224, 224)
    recon = (acc[0, 0] - acc[0, 1]) * jnp.float32(1.0 / (B * 72 * _PIX))
    return x_hat, recon, vq_loss, jnp.stack(perps)


# speedup vs baseline: 1.9289x; 1.0971x over previous
import functools

import numpy as np
import jax
import jax.numpy as jnp
from jax.experimental import pallas as pl
from jax.experimental.pallas import tpu as pltpu

_SCALES = (1, 2, 4, 7, 14, 28)
_NC = 8192
_H = 28
_HW = _H * _H
_HIGH = jax.lax.Precision.HIGHEST


def _pool_A(s):
    k = _H // s
    P = np.zeros((s, _H), np.float32)
    for i in range(s):
        P[i, i * k:(i + 1) * k] = np.float32(1.0 / k)
    return np.einsum('iy,jx->ijyx', P, P).reshape(s * s, _HW).copy()


@functools.lru_cache(maxsize=None)
def _pool_mats():
    return {s: _pool_A(s) for s in _SCALES}


def _vq_chunk(tok, C, cn, Csplit):
    zn = jnp.sum(tok * tok, axis=1, keepdims=True)
    mm = jax.lax.dot_general(tok, C, (((1,), (1,)), ((), ())),
                             preferred_element_type=jnp.float32)
    d = (zn + cn) - 2.0 * mm
    m = jnp.min(d, axis=1, keepdims=True)
    iota = jax.lax.broadcasted_iota(jnp.int32, d.shape, 1)
    idx = jnp.min(jnp.where(d == m, iota, jnp.int32(_NC)), axis=1, keepdims=True)
    oh = (iota == idx).astype(jnp.float32)
    c1, c2, c3 = Csplit
    zq = jax.lax.dot_general(oh, c1, (((1,), (0,)), ((), ())),
                             preferred_element_type=jnp.float32)
    zq = zq + jax.lax.dot_general(oh, c2, (((1,), (0,)), ((), ())),
                                  preferred_element_type=jnp.float32)
    zq = zq + jax.lax.dot_general(oh, c3, (((1,), (0,)), ((), ())),
                                  preferred_element_type=jnp.float32)
    cnts = jax.lax.dot_general(jnp.ones((1, tok.shape[0]), jnp.float32), oh,
                               (((1,), (0,)), ((), ())),
                               preferred_element_type=jnp.float32)
    e = (zq - tok) ** 2
    sse = jnp.sum(jnp.sum(e, axis=1, keepdims=True), axis=0, keepdims=True)
    return zq, cnts, sse


def _vq_scale_body(s, f_ref, cb_ref, a_ref, zq_ref, vl_ref, perp_ref, tok_ref):
    T = s * s
    n = 2 * T
    C = cb_ref[...]
    cn = jax.lax.dot_general(jnp.ones((1, 128), jnp.float32), C * C,
                             (((1,), (1,)), ((), ())),
                             precision=_HIGH, preferred_element_type=jnp.float32)
    c1 = C.astype(jnp.bfloat16).astype(jnp.float32)
    r = C - c1
    c2 = r.astype(jnp.bfloat16).astype(jnp.float32)
    Csplit = (c1, c2, r - c2)
    A = a_ref[...]
    t0 = jnp.dot(A, f_ref[0], precision=_HIGH, preferred_element_type=jnp.float32)
    t1 = jnp.dot(A, f_ref[1], precision=_HIGH, preferred_element_type=jnp.float32)
    toks = jnp.concatenate([t0, t1], axis=0)

    if s == 28:
        tok_ref[...] = toks
        chunk = 224

        def body(j, carry):
            cnts_c, sse_c = carry
            tk = tok_ref[pl.ds(j * chunk, chunk), :]
            zq_c, c_c, s_c = _vq_chunk(tk, C, cn, Csplit)
            zq_ref[pl.ds(j * chunk, chunk), :] = zq_c
            return (cnts_c + c_c, sse_c + s_c)

        cnts, sse = jax.lax.fori_loop(
            0, n // chunk, body,
            (jnp.zeros((1, _NC), jnp.float32), jnp.zeros((1, 1), jnp.float32)))
    elif s == 14:
        h = n // 2
        zq_a, cnts_a, sse_a = _vq_chunk(toks[:h], C, cn, Csplit)
        zq_b, cnts_b, sse_b = _vq_chunk(toks[h:], C, cn, Csplit)
        zq_ref[...] = jnp.concatenate([zq_a, zq_b], axis=0)
        cnts = cnts_a + cnts_b
        sse = sse_a + sse_b
    else:
        zq_all, cnts, sse = _vq_chunk(toks, C, cn, Csplit)
        zq_ref[...] = zq_all

    avg = cnts * jnp.float32(1.0 / n)
    ent = jnp.sum(avg * jnp.log(avg + 1e-10), axis=1, keepdims=True)
    perp_ref[...] = jnp.exp(-ent)
    mse = sse * jnp.float32(1.0 / (n * 128))
    vl_ref[...] = mse + 0.25 * mse


@functools.lru_cache(maxsize=None)
def _vq_scale_call(s):
    T = s * s
    body = functools.partial(_vq_scale_body, s)
    return pl.pallas_call(
        body,
        out_shape=[
            jax.ShapeDtypeStruct((2 * T, 128), jnp.float32),
            jax.ShapeDtypeStruct((1, 1), jnp.float32),
            jax.ShapeDtypeStruct((1, 1), jnp.float32),
        ],
        scratch_shapes=[pltpu.VMEM((2 * T, 128), jnp.float32)],
    )


_PIX = 224 * 224
_TBLK = 3584


def _tail_body(h_ref, x_ref, w_ref, b_ref, xhat_ref, acc_ref):
    v = jax.lax.dot_general(w_ref[...], h_ref[0], (((1,), (0,)), ((), ())),
                            preferred_element_type=jnp.float32) + b_ref[...]
    xhat_ref[0] = v
    pw = jnp.maximum(v, 0.0) + jnp.log1p(jnp.exp(-jnp.abs(v)))
    psum = jnp.sum(jnp.sum(pw, axis=1, keepdims=True), axis=0, keepdims=True)
    chan = jax.lax.broadcasted_iota(jnp.int32, v.shape, 0)
    gsum = jnp.zeros((1, 1), jnp.float32)
    for z in range(4):
        lane = x_ref[0, z:z + 1, :] + jnp.int32(18 * z)
        sel = jnp.where(chan == lane, v, 0.0)
        gsum = gsum + jnp.sum(jnp.sum(sel, axis=1, keepdims=True), axis=0, keepdims=True)

    i = pl.program_id(0)
    j = pl.program_id(1)

    @pl.when(jnp.logical_and(i == 0, j == 0))
    def _():
        acc_ref[...] = jnp.zeros_like(acc_ref)

    acc_ref[...] = acc_ref[...] + jnp.concatenate([psum, gsum], axis=1)


@functools.lru_cache(maxsize=None)
def _tail_call():
    nblk = _PIX // _TBLK
    return pl.pallas_call(
        _tail_body,
        grid=(2, nblk),
        in_specs=[
            pl.BlockSpec((1, 64, _TBLK), lambda i, j: (i, 0, j)),

            pl.BlockSpec((1, 4, _TBLK), lambda i, j: (i, 0, j)),
            pl.BlockSpec((72, 64), lambda i, j: (0, 0)),
            pl.BlockSpec((72, 1), lambda i, j: (0, 0)),
        ],
        out_specs=[
            pl.BlockSpec((1, 72, _TBLK), lambda i, j: (i, 0, j)),
            pl.BlockSpec((1, 2), lambda i, j: (0, 0)),
        ],
        out_shape=[
            jax.ShapeDtypeStruct((2, 72, _PIX), jnp.float32),
            jax.ShapeDtypeStruct((1, 2), jnp.float32),
        ],
    )


def _conv2d(x, w, b=None, stride=1, pad=0):
    out = jax.lax.conv_general_dilated(x, w, (stride, stride), [(pad, pad), (pad, pad)],
                                       dimension_numbers=('NCHW', 'OIHW', 'NCHW'))
    if b is not None:
        out = out + b[None, :, None, None]
    return out


def _conv_transpose2d(x, w, b, stride=2, pad=1):
    k = w.shape[2]
    wt = jnp.flip(w, axis=(2, 3)).transpose(1, 0, 2, 3)
    e = k - 1 - pad
    out = jax.lax.conv_general_dilated(x, wt, (1, 1), [(e, e), (e, e)],
                                       lhs_dilation=(stride, stride),
                                       dimension_numbers=('NCHW', 'OIHW', 'NCHW'))
    return out + b[None, :, None, None]


def _bn_eval(x, g, b, eps=1e-5):
    return x / np.float32(np.sqrt(1.0 + eps)) * g[None, :, None, None] + b[None, :, None, None]


def _encoder(xf, p):
    h = _conv2d(xf, p['enc_comp_w'], None, 1, 1)
    h = jax.nn.relu(_bn_eval(h, p['enc_bn_g'], p['enc_bn_b']))
    h = jax.nn.relu(_conv2d(h, p['enc_w1'], p['enc_b1'], 2, 1))
    h = jax.nn.relu(_conv2d(h, p['enc_w2'], p['enc_b2'], 2, 1))
    h = jax.nn.relu(_conv2d(h, p['enc_w3'], p['enc_b3'], 2, 1))
    h = _conv2d(h, p['enc_w4'], p['enc_b4'], 1, 1)
    return h


def kernel(x, params):
    p = params
    B = x.shape[0]
    x_oh = jax.nn.one_hot(x, 18, dtype=jnp.float32)
    xf = x_oh.transpose(0, 1, 4, 2, 3).reshape(B, x.shape[1] * 18, x.shape[2], x.shape[3])
    f = _encoder(xf, p)
    cb = p['codebook']

    with jax.ensure_compile_time_eval():
        A_mats = _pool_mats()

    vq_loss = jnp.asarray(0.0, jnp.float32)
    perps = []
    fd = jnp.zeros_like(f)
    for i, s in enumerate(_SCALES):
        f_tok = jnp.transpose(f, (0, 2, 3, 1)).reshape(B, _HW, 128)
        zq, vl, perp = _vq_scale_call(s)(f_tok, cb, jnp.asarray(A_mats[s]))
        vq_loss = vq_loss + vl[0, 0]
        perps.append(perp[0, 0])
        z = zq.reshape(B, s, s, 128).transpose(0, 3, 1, 2)
        z = jax.image.resize(z, (B, 128, _H, _H), method='cubic')
        f = f - jax.nn.relu(_conv2d(z, p['phi_enc_w'][i], p['phi_enc_b'][i], 1, 1))
        fd = fd + jax.nn.relu(_conv2d(z, p['phi_dec_w'][i], p['phi_dec_b'][i], 1, 1))

    h = jax.nn.relu(_conv2d(fd, p['dec_w0'], p['dec_b0'], 1, 1))
    h = jax.nn.relu(_conv_transpose2d(h, p['dct_w1'], p['dct_b1']))
    h = jax.nn.relu(_conv_transpose2d(h, p['dct_w2'], p['dct_b2']))
    h = _conv_transpose2d(h, p['dct_w3'], p['dct_b3'])

    w_oc = p['decomp_w'][:, :, 0, 0]
    h16 = h.astype(jnp.bfloat16)
    xh, acc = _tail_call()(h16.reshape(B, 64, _PIX), x.reshape(B, 4, _PIX),
                           w_oc, p['decomp_b'][:, None])
    x_hat = xh.reshape(B, 72, 224, 224)
    recon = (acc[0, 0] - acc[0, 1]) * jnp.float32(1.0 / (B * 72 * _PIX))
    return x_hat, recon, vq_loss, jnp.stack(perps)


# speedup vs baseline: 1.9938x; 1.0336x over previous
import functools

import numpy as np
import jax
import jax.numpy as jnp
from jax.experimental import pallas as pl
from jax.experimental.pallas import tpu as pltpu

_SCALES = (1, 2, 4, 7, 14, 28)
_NC = 8192
_H = 28
_HW = _H * _H
_HIGH = jax.lax.Precision.HIGHEST


def _pool_A(s):
    k = _H // s
    P = np.zeros((s, _H), np.float32)
    for i in range(s):
        P[i, i * k:(i + 1) * k] = np.float32(1.0 / k)
    return np.einsum('iy,jx->ijyx', P, P).reshape(s * s, _HW).copy()


@functools.lru_cache(maxsize=None)
def _pool_mats():
    return {s: (_pool_A(s) if s != _H else np.zeros((8, 128), np.float32))
            for s in _SCALES}


def _vq_chunk(tok, C, cn, Csplit):
    zn = jnp.sum(tok * tok, axis=1, keepdims=True)
    mm = jax.lax.dot_general(tok, C, (((1,), (1,)), ((), ())),
                             preferred_element_type=jnp.float32)
    d = (zn + cn) - 2.0 * mm
    m = jnp.min(d, axis=1, keepdims=True)
    iota = jax.lax.broadcasted_iota(jnp.int32, d.shape, 1)
    idx = jnp.min(jnp.where(d == m, iota, jnp.int32(_NC)), axis=1, keepdims=True)
    oh = (iota == idx).astype(jnp.bfloat16)
    c1, c2, c3 = Csplit
    zq = jax.lax.dot_general(oh, c1, (((1,), (0,)), ((), ())),
                             preferred_element_type=jnp.float32)
    zq = zq + jax.lax.dot_general(oh, c2, (((1,), (0,)), ((), ())),
                                  preferred_element_type=jnp.float32)
    zq = zq + jax.lax.dot_general(oh, c3, (((1,), (0,)), ((), ())),
                                  preferred_element_type=jnp.float32)
    cnts = jax.lax.dot_general(jnp.ones((1, tok.shape[0]), jnp.bfloat16), oh,
                               (((1,), (0,)), ((), ())),
                               preferred_element_type=jnp.float32)
    e = (zq - tok) ** 2
    sse = jnp.sum(jnp.sum(e, axis=1, keepdims=True), axis=0, keepdims=True)
    return zq, cnts, sse


def _vq_scale_body(s, f_ref, cb_ref, a_ref, zq_ref, vl_ref, perp_ref, tok_ref):
    T = s * s
    n = 2 * T
    C = cb_ref[...]
    cn = jax.lax.dot_general(jnp.ones((1, 128), jnp.float32), C * C,
                             (((1,), (1,)), ((), ())),
                             precision=_HIGH, preferred_element_type=jnp.float32)
    c1 = C.astype(jnp.bfloat16).astype(jnp.float32)
    r = C - c1
    c2 = r.astype(jnp.bfloat16).astype(jnp.float32)
    Csplit = (c1, c2, r - c2)
    if s == _H:
        toks = jnp.concatenate([f_ref[0], f_ref[1]], axis=0)
    else:
        A = a_ref[...]
        t0 = jnp.dot(A, f_ref[0], precision=_HIGH, preferred_element_type=jnp.float32)
        t1 = jnp.dot(A, f_ref[1], precision=_HIGH, preferred_element_type=jnp.float32)
        toks = jnp.concatenate([t0, t1], axis=0)

    if s == 28:
        tok_ref[...] = toks
        chunk = 224

        def body(j, carry):
            cnts_c, sse_c = carry
            tk = tok_ref[pl.ds(j * chunk, chunk), :]
            zq_c, c_c, s_c = _vq_chunk(tk, C, cn, Csplit)
            zq_ref[pl.ds(j * chunk, chunk), :] = zq_c
            return (cnts_c + c_c, sse_c + s_c)

        cnts, sse = jax.lax.fori_loop(
            0, n // chunk, body,
            (jnp.zeros((1, _NC), jnp.float32), jnp.zeros((1, 1), jnp.float32)))
    elif s == 14:
        h = n // 2
        zq_a, cnts_a, sse_a = _vq_chunk(toks[:h], C, cn, Csplit)
        zq_b, cnts_b, sse_b = _vq_chunk(toks[h:], C, cn, Csplit)
        zq_ref[...] = jnp.concatenate([zq_a, zq_b], axis=0)
        cnts = cnts_a + cnts_b
        sse = sse_a + sse_b
    else:
        zq_all, cnts, sse = _vq_chunk(toks, C, cn, Csplit)
        zq_ref[...] = zq_all

    avg = cnts * jnp.float32(1.0 / n)
    ent = jnp.sum(avg * jnp.log(avg + 1e-10), axis=1, keepdims=True)
    perp_ref[...] = jnp.exp(-ent)
    mse = sse * jnp.float32(1.0 / (n * 128))
    vl_ref[...] = mse + 0.25 * mse


@functools.lru_cache(maxsize=None)
def _vq_scale_call(s):
    T = s * s
    body = functools.partial(_vq_scale_body, s)
    return pl.pallas_call(
        body,
        out_shape=[
            jax.ShapeDtypeStruct((2 * T, 128), jnp.float32),
            jax.ShapeDtypeStruct((1, 1), jnp.float32),
            jax.ShapeDtypeStruct((1, 1), jnp.float32),
        ],
        scratch_shapes=[pltpu.VMEM((2 * T, 128), jnp.float32)],
    )


_PIX = 224 * 224
_TBLK = 3584


def _tail_body(h_ref, x_ref, w_ref, b_ref, xhat_ref, acc_ref):
    v = jax.lax.dot_general(w_ref[...], h_ref[0], (((1,), (0,)), ((), ())),
                            preferred_element_type=jnp.float32) + b_ref[...]
    xhat_ref[0] = v
    pw = jnp.maximum(v, 0.0) + jnp.log1p(jnp.exp(-jnp.abs(v)))
    psum = jnp.sum(jnp.sum(pw, axis=1, keepdims=True), axis=0, keepdims=True)
    chan = jax.lax.broadcasted_iota(jnp.int32, v.shape, 0)
    gsum = jnp.zeros((1, 1), jnp.float32)
    for z in range(4):
        lane = x_ref[0, z:z + 1, :] + jnp.int32(18 * z)
        sel = jnp.where(chan == lane, v, 0.0)
        gsum = gsum + jnp.sum(jnp.sum(sel, axis=1, keepdims=True), axis=0, keepdims=True)

    i = pl.program_id(0)
    j = pl.program_id(1)

    @pl.when(jnp.logical_and(i == 0, j == 0))
    def _():
        acc_ref[...] = jnp.zeros_like(acc_ref)

    acc_ref[...] = acc_ref[...] + jnp.concatenate([psum, gsum], axis=1)


@functools.lru_cache(maxsize=None)
def _tail_call():
    nblk = _PIX // _TBLK
    return pl.pallas_call(
        _tail_body,
        grid=(2, nblk),
        in_specs=[
            pl.BlockSpec((1, 64, _TBLK), lambda i, j: (i, 0, j)),

            pl.BlockSpec((1, 4, _TBLK), lambda i, j: (i, 0, j)),
            pl.BlockSpec((72, 64), lambda i, j: (0, 0)),
            pl.BlockSpec((72, 1), lambda i, j: (0, 0)),
        ],
        out_specs=[
            pl.BlockSpec((1, 72, _TBLK), lambda i, j: (i, 0, j)),
            pl.BlockSpec((1, 2), lambda i, j: (0, 0)),
        ],
        out_shape=[
            jax.ShapeDtypeStruct((2, 72, _PIX), jnp.float32),
            jax.ShapeDtypeStruct((1, 2), jnp.float32),
        ],
    )


def _conv2d(x, w, b=None, stride=1, pad=0):
    out = jax.lax.conv_general_dilated(x, w, (stride, stride), [(pad, pad), (pad, pad)],
                                       dimension_numbers=('NCHW', 'OIHW', 'NCHW'))
    if b is not None:
        out = out + b[None, :, None, None]
    return out


def _conv_transpose2d(x, w, b, stride=2, pad=1):
    k = w.shape[2]
    wt = jnp.flip(w, axis=(2, 3)).transpose(1, 0, 2, 3)
    e = k - 1 - pad
    out = jax.lax.conv_general_dilated(x, wt, (1, 1), [(e, e), (e, e)],
                                       lhs_dilation=(stride, stride),
                                       dimension_numbers=('NCHW', 'OIHW', 'NCHW'))
    return out + b[None, :, None, None]


def _bn_eval(x, g, b, eps=1e-5):
    return x / np.float32(np.sqrt(1.0 + eps)) * g[None, :, None, None] + b[None, :, None, None]


def _encoder(xf, p):
    h = _conv2d(xf, p['enc_comp_w'], None, 1, 1)
    h = jax.nn.relu(_bn_eval(h, p['enc_bn_g'], p['enc_bn_b']))
    h = jax.nn.relu(_conv2d(h, p['enc_w1'], p['enc_b1'], 2, 1))
    h = jax.nn.relu(_conv2d(h, p['enc_w2'], p['enc_b2'], 2, 1))
    h = jax.nn.relu(_conv2d(h, p['enc_w3'], p['enc_b3'], 2, 1))
    h = _conv2d(h, p['enc_w4'], p['enc_b4'], 1, 1)
    return h


def kernel(x, params):
    p = params
    B = x.shape[0]
    x_oh = jax.nn.one_hot(x, 18, dtype=jnp.float32)
    xf = x_oh.transpose(0, 1, 4, 2, 3).reshape(B, x.shape[1] * 18, x.shape[2], x.shape[3])
    f = _encoder(xf, p)
    cb = p['codebook']

    with jax.ensure_compile_time_eval():
        A_mats = _pool_mats()

    vq_loss = jnp.asarray(0.0, jnp.float32)
    perps = []
    fd = jnp.zeros_like(f)
    for i, s in enumerate(_SCALES):
        f_tok = jnp.transpose(f, (0, 2, 3, 1)).reshape(B, _HW, 128)
        zq, vl, perp = _vq_scale_call(s)(f_tok, cb, jnp.asarray(A_mats[s]))
        vq_loss = vq_loss + vl[0, 0]
        perps.append(perp[0, 0])
        z = zq.reshape(B, s, s, 128).transpose(0, 3, 1, 2)
        z = jax.image.resize(z, (B, 128, _H, _H), method='cubic')
        f = f - jax.nn.relu(_conv2d(z, p['phi_enc_w'][i], p['phi_enc_b'][i], 1, 1))
        fd = fd + jax.nn.relu(_conv2d(z, p['phi_dec_w'][i], p['phi_dec_b'][i], 1, 1))

    h = jax.nn.relu(_conv2d(fd, p['dec_w0'], p['dec_b0'], 1, 1))
    h = jax.nn.relu(_conv_transpose2d(h, p['dct_w1'], p['dct_b1']))
    h = jax.nn.relu(_conv_transpose2d(h, p['dct_w2'], p['dct_b2']))
    h = _conv_transpose2d(h, p['dct_w3'], p['dct_b3'])

    w_oc = p['decomp_w'][:, :, 0, 0]
    h16 = h.astype(jnp.bfloat16)
    xh, acc = _tail_call()(h16.reshape(B, 64, _PIX), x.reshape(B, 4, _PIX),
                           w_oc, p['decomp_b'][:, None])
    x_hat = xh.reshape(B, 72, 224, 224)
    recon = (acc[0, 0] - acc[0, 1]) * jnp.float32(1.0 / (B * 72 * _PIX))
    return x_hat, recon, vq_loss, jnp.stack(perps)
